# baseline (speedup 1.0000x reference)
import functools

import jax
import jax.numpy as jnp
from jax import lax
from jax.experimental import pallas as pl
from jax.experimental.pallas import tpu as pltpu
from jax.experimental.pallas import tpu_sc as plsc

S = 32
B = 16384
NW = 32
BPW = B // NW
NROW = S + 1
NIDX = NROW * BPW
CHUNK = 128
NCH = NIDX // CHUNK
LN2 = 0.6931471805599453


def _sc_body(idx_hbm, w_hbm, out_hbm,
             i0r, i1r, i2r, i3r, i4r, idxb, gat, outv, sem):
    wid = lax.axis_index("s") * 2 + lax.axis_index("c")
    base_b = wid * BPW

    for r, ref in enumerate((i0r, i1r, i2r, i3r, i4r)):
        pltpu.sync_copy(idx_hbm.at[pl.ds(r * B + base_b, BPW)], ref)

    def fill(oc, _):
        sl = pl.ds(oc * 16, 16)
        i1v = i1r[sl]
        i2v = i2r[sl]
        i3v = i3r[sl]
        i4v = i4r[sl]
        offv = ((i1v * S + i2v) * S + i3v) * S + i4v

        def fill_s(s1, _):
            idxb[pl.ds(s1 * BPW + oc * 16, 16)] = offv + s1 * (1 << 20)
            return 0

        lax.fori_loop(0, S, fill_s, 0)
        i0v = i0r[sl]
        idxb[pl.ds(S * BPW + oc * 16, 16)] = offv + i0v * (1 << 20)
        return 0

    lax.fori_loop(0, BPW // 16, fill, 0)

    def fire(c, _):
        pltpu.async_copy(w_hbm.at[idxb.at[pl.ds(c * CHUNK, CHUNK)]],
                         gat.at[pl.ds(c * CHUNK, CHUNK)], sem)
        return 0

    lax.fori_loop(0, NCH, fire, 0)
    pltpu.make_async_copy(w_hbm.at[pl.ds(0, NIDX)], gat, sem).wait()

    def comp(oc, _):
        base = oc * 16

        def mx(s1, m):
            return jnp.maximum(m, gat[pl.ds(s1 * BPW + base, 16)])

        m = lax.fori_loop(1, S, mx, gat[pl.ds(base, 16)])

        def sm(s1, acc):
            return acc + jnp.exp(gat[pl.ds(s1 * BPW + base, 16)] - m)

        s = lax.fori_loop(0, S, sm, jnp.zeros((16,), jnp.float32))

        bi = lax.bitcast_convert_type(s, jnp.int32)
        e = lax.shift_right_logical(bi, 23) - 127
        mant = lax.bitcast_convert_type(
            (bi & 0x007FFFFF) | 0x3F800000, jnp.float32)
        big = mant > 1.4142135623730951
        mant = jnp.where(big, mant * 0.5, mant)
        e = jnp.where(big, e + 1, e)
        t = (mant - 1.0) / (mant + 1.0)
        t2 = t * t
        poly = 1.0 + t2 * (1.0 / 3.0 + t2 * (1.0 / 5.0 + t2 * (
            1.0 / 7.0 + t2 * (1.0 / 9.0))))
        logs = 2.0 * t * poly + e.astype(jnp.float32) * LN2
        lse = m + logs
        outv[pl.ds(base, 16)] = gat[pl.ds(S * BPW + base, 16)] - lse
        return 0

    lax.fori_loop(0, BPW // 16, comp, 0)

    pltpu.sync_copy(outv, out_hbm.at[pl.ds(base_b, BPW)])


_sc_call = functools.partial(
    pl.kernel,
    out_type=jax.ShapeDtypeStruct((B,), jnp.float32),
    mesh=plsc.VectorSubcoreMesh(core_axis_name="c", subcore_axis_name="s"),
    scratch_types=[
        pltpu.VMEM((BPW,), jnp.int32),
        pltpu.VMEM((BPW,), jnp.int32),
        pltpu.VMEM((BPW,), jnp.int32),
        pltpu.VMEM((BPW,), jnp.int32),
        pltpu.VMEM((BPW,), jnp.int32),
        pltpu.VMEM((NIDX,), jnp.int32),
        pltpu.VMEM((NIDX,), jnp.float32),
        pltpu.VMEM((BPW,), jnp.float32),
        pltpu.SemaphoreType.DMA,
    ],
)(_sc_body)


def kernel(idx, w):
    idxf = idx.reshape(-1)
    wf = w.reshape(-1)
    return _sc_call(idxf, wf)
